# baseline (speedup 1.0000x reference)
import functools

import jax
import jax.numpy as jnp
from jax import lax
from jax.experimental import pallas as pl
from jax.experimental.pallas import tpu as pltpu
from jax.experimental.pallas import tpu_sc as plsc

B, T, D = 4, 2048, 1024
E, TOPK = 8, 2
N = B * T
CAP = N * TOPK // E
R = E * CAP
RPAD = 8
HID = 4 * D
QW = 128

TB = 1024
MB = 1024
FB = 2048
NM = CAP // MB
NF = HID // FB


def _router_body(x_ref, lwwT_ref, lwb_ref, lnwT_ref, lnb_ref, noise_ref,
                 dst0_ref, dst1_ref, q0_ref, q1_ref, cnt_ref):
    pid = pl.program_id(0)

    @pl.when(pid == 0)
    def _():
        cnt_ref[...] = jnp.zeros_like(cnt_ref)

    x = x_ref[...]
    logits = jnp.dot(x, lwwT_ref[...], preferred_element_type=jnp.float32)
    logits = logits + lwb_ref[...]
    zn = jnp.dot(x, lnwT_ref[...], preferred_element_type=jnp.float32)
    zn = zn + lnb_ref[...]
    sp = jnp.maximum(zn, 0.0) + jnp.log1p(jnp.exp(-jnp.abs(zn)))
    noisy = logits + noise_ref[...] * sp

    iota_e = lax.broadcasted_iota(jnp.int32, (TB, E), 1)
    v0 = jnp.max(noisy, axis=1, keepdims=True)
    e0 = jnp.min(jnp.where(noisy == v0, iota_e, E), axis=1, keepdims=True)
    oh0 = iota_e == e0
    noisy2 = jnp.where(oh0, -jnp.inf, noisy)
    v1 = jnp.max(noisy2, axis=1, keepdims=True)
    e1 = jnp.min(jnp.where(noisy2 == v1, iota_e, E), axis=1, keepdims=True)
    oh1 = iota_e == e1

    ex = jnp.exp(v1 - v0)
    denom = 1.0 + ex
    p0 = 1.0 / denom
    p1 = ex / denom

    m = (oh0 | oh1).astype(jnp.float32)
    ii = lax.broadcasted_iota(jnp.int32, (TB, TB), 0)
    jj = lax.broadcasted_iota(jnp.int32, (TB, TB), 1)
    lt = (jj < ii).astype(jnp.float32)
    excl = jnp.dot(lt, m, preferred_element_type=jnp.float32) + cnt_ref[...]
    pos0 = jnp.sum(jnp.where(oh0, excl, 0.0), axis=1, keepdims=True)
    pos1 = jnp.sum(jnp.where(oh1, excl, 0.0), axis=1, keepdims=True)
    cnt_ref[...] = cnt_ref[...] + jnp.sum(m, axis=0, keepdims=True)

    pos0 = pos0.astype(jnp.int32)
    pos1 = pos1.astype(jnp.int32)
    keep0 = pos0 < CAP
    keep1 = pos1 < CAP
    dst0_ref[...] = jnp.where(keep0, e0 * CAP + pos0, R)
    dst1_ref[...] = jnp.where(keep1, e1 * CAP + pos1, R)
    q0_ref[...] = jnp.where(keep0, p0, 0.0)
    q1_ref[...] = jnp.where(keep1, p1, 0.0)


def _router(x2d, noise2d, lwwT, lwb, lnwT, lnb):
    nblk = N // TB
    out = pl.pallas_call(
        _router_body,
        grid=(nblk,),
        in_specs=[
            pl.BlockSpec((TB, D), lambda i: (i, 0)),
            pl.BlockSpec((D, E), lambda i: (0, 0)),
            pl.BlockSpec((1, E), lambda i: (0, 0)),
            pl.BlockSpec((D, E), lambda i: (0, 0)),
            pl.BlockSpec((1, E), lambda i: (0, 0)),
            pl.BlockSpec((TB, E), lambda i: (i, 0)),
        ],
        out_specs=[
            pl.BlockSpec((TB, 1), lambda i: (i, 0)),
            pl.BlockSpec((TB, 1), lambda i: (i, 0)),
            pl.BlockSpec((TB, 1), lambda i: (i, 0)),
            pl.BlockSpec((TB, 1), lambda i: (i, 0)),
        ],
        out_shape=[
            jax.ShapeDtypeStruct((N, 1), jnp.int32),
            jax.ShapeDtypeStruct((N, 1), jnp.int32),
            jax.ShapeDtypeStruct((N, 1), jnp.float32),
            jax.ShapeDtypeStruct((N, 1), jnp.float32),
        ],
        scratch_shapes=[pltpu.VMEM((1, E), jnp.float32)],
    )(x2d, lwwT, lwb, lnwT, lnb, noise2d)
    dst0, dst1, q0, q1 = out
    return (dst0.reshape(N), dst1.reshape(N), q0.reshape(N), q1.reshape(N))


def _dispatch(x2d, dst0, dst1, q0, q1):
    mesh = plsc.VectorSubcoreMesh(core_axis_name="c", subcore_axis_name="s")
    nw = mesh.num_cores * mesh.num_subcores
    per_w = N // nw
    ch = 64
    nch = per_w // ch

    @functools.partial(
        pl.kernel,
        out_type=(
            jax.ShapeDtypeStruct((R + MB, D), jnp.float32),
            jax.ShapeDtypeStruct((R + MB, QW), jnp.float32),
        ),
        mesh=mesh,
        scratch_types=[
            pltpu.VMEM((ch,), jnp.int32),
            pltpu.VMEM((ch,), jnp.int32),
            pltpu.VMEM((ch,), jnp.float32),
            pltpu.VMEM((ch,), jnp.float32),
            pltpu.VMEM((ch, D), jnp.float32),
            pltpu.VMEM((ch, QW), jnp.float32),
            pltpu.VMEM((ch, QW), jnp.float32),
            pltpu.SemaphoreType.DMA,
            pltpu.SemaphoreType.DMA,
        ],
    )
    def disp(x_hbm, d0_hbm, d1_hbm, q0_hbm, q1_hbm, xe_hbm, qxe_hbm,
             idx0_v, idx1_v, q0_v, q1_v, xbuf, qb0, qb1, sem0, sem1):
        wid = lax.axis_index("s") * mesh.num_cores + lax.axis_index("c")

        def body(ci, carry):
            base = pl.multiple_of(wid * per_w + ci * ch, ch)
            pltpu.sync_copy(x_hbm.at[pl.ds(base, ch)], xbuf)
            pltpu.sync_copy(d0_hbm.at[pl.ds(base, ch)], idx0_v)
            pltpu.sync_copy(d1_hbm.at[pl.ds(base, ch)], idx1_v)
            pltpu.sync_copy(q0_hbm.at[pl.ds(base, ch)], q0_v)
            pltpu.sync_copy(q1_hbm.at[pl.ds(base, ch)], q1_v)
            for g in range(ch // 16):
                qv0 = q0_v[pl.ds(g * 16, 16)]
                qv1 = q1_v[pl.ds(g * 16, 16)]
                for t16 in range(16):
                    t = g * 16 + t16
                    v0b = jnp.full((16,), qv0[t16], jnp.float32)
                    v1b = jnp.full((16,), qv1[t16], jnp.float32)
                    for j in range(QW // 16):
                        qb0[t, pl.ds(j * 16, 16)] = v0b
                        qb1[t, pl.ds(j * 16, 16)] = v1b
            c0 = pltpu.async_copy(xbuf, xe_hbm.at[idx0_v], sem0)
            c1 = pltpu.async_copy(xbuf, xe_hbm.at[idx1_v], sem1)
            c2 = pltpu.async_copy(qb0, qxe_hbm.at[idx0_v], sem0)
            c3 = pltpu.async_copy(qb1, qxe_hbm.at[idx1_v], sem1)
            c0.wait()
            c1.wait()
            c2.wait()
            c3.wait()
            return carry

        lax.fori_loop(0, nch, body, 0)

    return disp(x2d, dst0, dst1, q0, q1)


def _ffn_body(xe_ref, w1_ref, b1_ref, w2_ref, b2_ref, q_ref, out_ref):
    f = pl.program_id(1)
    xb = xe_ref[...].astype(jnp.bfloat16)
    h = lax.dot_general(xb, w1_ref[0], (((1,), (1,)), ((), ())),
                        preferred_element_type=jnp.float32)
    h = jnp.maximum(h + b1_ref[0], 0.0).astype(jnp.bfloat16)
    part = lax.dot_general(h, w2_ref[0], (((1,), (1,)), ((), ())),
                           preferred_element_type=jnp.float32)

    @pl.when(f == 0)
    def _():
        out_ref[...] = part + b2_ref[0]

    @pl.when(jnp.logical_and(f > 0, f < NF - 1))
    def _():
        out_ref[...] = out_ref[...] + part

    @pl.when(f == NF - 1)
    def _():
        out_ref[...] = (out_ref[...] + part) * q_ref[:, 0:1]


def _ffn(xe, qxe, W1, b1, W2, b2):
    nb = R // MB + 1

    def eidx(b):
        return jnp.minimum(b // NM, E - 1)

    return pl.pallas_call(
        _ffn_body,
        grid=(nb, NF),
        in_specs=[
            pl.BlockSpec((MB, D), lambda b, f: (b, 0)),
            pl.BlockSpec((1, FB, D), lambda b, f: (eidx(b), f, 0)),
            pl.BlockSpec((1, 1, FB), lambda b, f: (eidx(b) * NF + f, 0, 0)),
            pl.BlockSpec((1, D, FB), lambda b, f: (eidx(b), 0, f)),
            pl.BlockSpec((1, 1, D), lambda b, f: (eidx(b), 0, 0)),
            pl.BlockSpec((MB, QW), lambda b, f: (b, 0)),
        ],
        out_specs=pl.BlockSpec((MB, D), lambda b, f: (b, 0)),
        out_shape=jax.ShapeDtypeStruct((R + MB, D), jnp.float32),
        compiler_params=pltpu.CompilerParams(
            dimension_semantics=("parallel", "arbitrary")),
    )(xe, W1.astype(jnp.bfloat16), b1.reshape(E * NF, 1, FB),
      W2.astype(jnp.bfloat16), b2.reshape(E, 1, D), qxe)


def _combine(outbuf, dst0, dst1):
    mesh = plsc.VectorSubcoreMesh(core_axis_name="c", subcore_axis_name="s")
    nw = mesh.num_cores * mesh.num_subcores
    per_w = N // nw
    ch = 32
    nch = per_w // ch
    nv = D // 16

    @functools.partial(
        pl.kernel,
        out_type=jax.ShapeDtypeStruct((N, D), jnp.float32),
        mesh=mesh,
        scratch_types=[
            pltpu.VMEM((ch,), jnp.int32),
            pltpu.VMEM((ch,), jnp.int32),
            pltpu.VMEM((ch, D), jnp.float32),
            pltpu.VMEM((ch, D), jnp.float32),
            pltpu.SemaphoreType.DMA,
            pltpu.SemaphoreType.DMA,
        ],
    )
    def comb(ob_hbm, d0_hbm, d1_hbm, y_hbm,
             idx0_v, idx1_v, r0_v, r1_v, sem0, sem1):
        wid = lax.axis_index("s") * mesh.num_cores + lax.axis_index("c")

        def body(ci, carry):
            base = pl.multiple_of(wid * per_w + ci * ch, ch)
            pltpu.sync_copy(d0_hbm.at[pl.ds(base, ch)], idx0_v)
            pltpu.sync_copy(d1_hbm.at[pl.ds(base, ch)], idx1_v)
            c0 = pltpu.async_copy(ob_hbm.at[idx0_v], r0_v, sem0)
            c1 = pltpu.async_copy(ob_hbm.at[idx1_v], r1_v, sem1)
            c0.wait()
            c1.wait()

            def tok(t, carry2):
                def vec(v, c):
                    sl = pl.ds(v * 16, 16)
                    r0_v[t, sl] = r0_v[t, sl] + r1_v[t, sl]
                    return c
                return lax.fori_loop(0, nv, vec, carry2, unroll=8)

            lax.fori_loop(0, ch, tok, 0)
            pltpu.sync_copy(r0_v, y_hbm.at[pl.ds(base, ch)])
            return carry

        lax.fori_loop(0, nch, body, 0)

    return comb(outbuf, dst0, dst1)


def kernel(x, noise, lw_w, lw_b, ln_w, ln_b, W1, b1, W2, b2):
    x2d = x.reshape(N, D)
    noise2d = noise.reshape(N, E)
    dst0, dst1, q0, q1 = _router(
        x2d, noise2d, lw_w.T, lw_b.reshape(1, E), ln_w.T, ln_b.reshape(1, E))
    xe, qxe = _dispatch(x2d, dst0, dst1, q0, q1)
    outbuf = _ffn(xe, qxe, W1, b1, W2, b2)
    y = _combine(outbuf, dst0, dst1)
    return y.reshape(B, T, D)


# speedup vs baseline: 1.0483x; 1.0483x over previous
import functools

import jax
import jax.numpy as jnp
from jax import lax
from jax.experimental import pallas as pl
from jax.experimental.pallas import tpu as pltpu
from jax.experimental.pallas import tpu_sc as plsc

B, T, D = 4, 2048, 1024
E, TOPK = 8, 2
N = B * T
CAP = N * TOPK // E
R = E * CAP
RPAD = 8
HID = 4 * D
QW = 128

TB = 1024
MB = 2048
FB = 512
NM = CAP // MB
NF = HID // FB


def _router_body(x_ref, lwwT_ref, lwb_ref, lnwT_ref, lnb_ref, noise_ref,
                 dst0_ref, dst1_ref, q0_ref, q1_ref, cnt_ref):
    pid = pl.program_id(0)

    @pl.when(pid == 0)
    def _():
        cnt_ref[...] = jnp.zeros_like(cnt_ref)

    x = x_ref[...]
    logits = jnp.dot(x, lwwT_ref[...], preferred_element_type=jnp.float32)
    logits = logits + lwb_ref[...]
    zn = jnp.dot(x, lnwT_ref[...], preferred_element_type=jnp.float32)
    zn = zn + lnb_ref[...]
    sp = jnp.maximum(zn, 0.0) + jnp.log1p(jnp.exp(-jnp.abs(zn)))
    noisy = logits + noise_ref[...] * sp

    iota_e = lax.broadcasted_iota(jnp.int32, (TB, E), 1)
    v0 = jnp.max(noisy, axis=1, keepdims=True)
    e0 = jnp.min(jnp.where(noisy == v0, iota_e, E), axis=1, keepdims=True)
    oh0 = iota_e == e0
    noisy2 = jnp.where(oh0, -jnp.inf, noisy)
    v1 = jnp.max(noisy2, axis=1, keepdims=True)
    e1 = jnp.min(jnp.where(noisy2 == v1, iota_e, E), axis=1, keepdims=True)
    oh1 = iota_e == e1

    ex = jnp.exp(v1 - v0)
    denom = 1.0 + ex
    p0 = 1.0 / denom
    p1 = ex / denom

    m = (oh0 | oh1).astype(jnp.float32)
    ii = lax.broadcasted_iota(jnp.int32, (TB, TB), 0)
    jj = lax.broadcasted_iota(jnp.int32, (TB, TB), 1)
    lt = (jj < ii).astype(jnp.float32)
    excl = jnp.dot(lt, m, preferred_element_type=jnp.float32) + cnt_ref[...]
    pos0 = jnp.sum(jnp.where(oh0, excl, 0.0), axis=1, keepdims=True)
    pos1 = jnp.sum(jnp.where(oh1, excl, 0.0), axis=1, keepdims=True)
    cnt_ref[...] = cnt_ref[...] + jnp.sum(m, axis=0, keepdims=True)

    pos0 = pos0.astype(jnp.int32)
    pos1 = pos1.astype(jnp.int32)
    keep0 = pos0 < CAP
    keep1 = pos1 < CAP
    dst0_ref[...] = jnp.where(keep0, e0 * CAP + pos0, R)
    dst1_ref[...] = jnp.where(keep1, e1 * CAP + pos1, R)
    q0_ref[...] = jnp.where(keep0, p0, 0.0)
    q1_ref[...] = jnp.where(keep1, p1, 0.0)


def _router(x2d, noise2d, lwwT, lwb, lnwT, lnb):
    nblk = N // TB
    out = pl.pallas_call(
        _router_body,
        grid=(nblk,),
        in_specs=[
            pl.BlockSpec((TB, D), lambda i: (i, 0)),
            pl.BlockSpec((D, E), lambda i: (0, 0)),
            pl.BlockSpec((1, E), lambda i: (0, 0)),
            pl.BlockSpec((D, E), lambda i: (0, 0)),
            pl.BlockSpec((1, E), lambda i: (0, 0)),
            pl.BlockSpec((TB, E), lambda i: (i, 0)),
        ],
        out_specs=[
            pl.BlockSpec((TB, 1), lambda i: (i, 0)),
            pl.BlockSpec((TB, 1), lambda i: (i, 0)),
            pl.BlockSpec((TB, 1), lambda i: (i, 0)),
            pl.BlockSpec((TB, 1), lambda i: (i, 0)),
        ],
        out_shape=[
            jax.ShapeDtypeStruct((N, 1), jnp.int32),
            jax.ShapeDtypeStruct((N, 1), jnp.int32),
            jax.ShapeDtypeStruct((N, 1), jnp.float32),
            jax.ShapeDtypeStruct((N, 1), jnp.float32),
        ],
        scratch_shapes=[pltpu.VMEM((1, E), jnp.float32)],
    )(x2d, lwwT, lwb, lnwT, lnb, noise2d)
    dst0, dst1, q0, q1 = out
    return (dst0.reshape(N), dst1.reshape(N), q0.reshape(N), q1.reshape(N))


def _dispatch(x2d, dst0, dst1, q0, q1):
    mesh = plsc.VectorSubcoreMesh(core_axis_name="c", subcore_axis_name="s")
    nw = mesh.num_cores * mesh.num_subcores
    per_w = N // nw
    ch = 64
    nch = per_w // ch

    @functools.partial(
        pl.kernel,
        out_type=(
            jax.ShapeDtypeStruct((R + MB, D), jnp.float32),
            jax.ShapeDtypeStruct((R + MB, QW), jnp.float32),
        ),
        mesh=mesh,
        scratch_types=[
            pltpu.VMEM((ch,), jnp.int32),
            pltpu.VMEM((ch,), jnp.int32),
            pltpu.VMEM((ch,), jnp.float32),
            pltpu.VMEM((ch,), jnp.float32),
            pltpu.VMEM((ch, D), jnp.float32),
            pltpu.VMEM((ch, QW), jnp.float32),
            pltpu.VMEM((ch, QW), jnp.float32),
            pltpu.SemaphoreType.DMA,
            pltpu.SemaphoreType.DMA,
        ],
    )
    def disp(x_hbm, d0_hbm, d1_hbm, q0_hbm, q1_hbm, xe_hbm, qxe_hbm,
             idx0_v, idx1_v, q0_v, q1_v, xbuf, qb0, qb1, sem0, sem1):
        wid = lax.axis_index("s") * mesh.num_cores + lax.axis_index("c")

        def body(ci, carry):
            base = pl.multiple_of(wid * per_w + ci * ch, ch)
            pltpu.sync_copy(x_hbm.at[pl.ds(base, ch)], xbuf)
            pltpu.sync_copy(d0_hbm.at[pl.ds(base, ch)], idx0_v)
            pltpu.sync_copy(d1_hbm.at[pl.ds(base, ch)], idx1_v)
            pltpu.sync_copy(q0_hbm.at[pl.ds(base, ch)], q0_v)
            pltpu.sync_copy(q1_hbm.at[pl.ds(base, ch)], q1_v)
            for g in range(ch // 16):
                qv0 = q0_v[pl.ds(g * 16, 16)]
                qv1 = q1_v[pl.ds(g * 16, 16)]
                for t16 in range(16):
                    t = g * 16 + t16
                    v0b = jnp.full((16,), qv0[t16], jnp.float32)
                    v1b = jnp.full((16,), qv1[t16], jnp.float32)
                    for j in range(QW // 16):
                        qb0[t, pl.ds(j * 16, 16)] = v0b
                        qb1[t, pl.ds(j * 16, 16)] = v1b
            c0 = pltpu.async_copy(xbuf, xe_hbm.at[idx0_v], sem0)
            c1 = pltpu.async_copy(xbuf, xe_hbm.at[idx1_v], sem1)
            c2 = pltpu.async_copy(qb0, qxe_hbm.at[idx0_v], sem0)
            c3 = pltpu.async_copy(qb1, qxe_hbm.at[idx1_v], sem1)
            c0.wait()
            c1.wait()
            c2.wait()
            c3.wait()
            return carry

        lax.fori_loop(0, nch, body, 0)

    return disp(x2d, dst0, dst1, q0, q1)


def _ffn_body(xe_ref, w1_ref, b1_ref, w2_ref, b2_ref, q_ref, out_ref):
    f = pl.program_id(1)
    xb = xe_ref[...]
    h = lax.dot_general(xb, w1_ref[0], (((1,), (1,)), ((), ())),
                        preferred_element_type=jnp.float32)
    h = jnp.maximum(h + b1_ref[0], 0.0)
    part = lax.dot_general(h, w2_ref[0], (((1,), (1,)), ((), ())),
                           preferred_element_type=jnp.float32)

    @pl.when(f == 0)
    def _():
        out_ref[...] = part + b2_ref[0]

    @pl.when(jnp.logical_and(f > 0, f < NF - 1))
    def _():
        out_ref[...] = out_ref[...] + part

    @pl.when(f == NF - 1)
    def _():
        out_ref[...] = (out_ref[...] + part) * q_ref[:, 0:1]


def _ffn(xe, qxe, W1, b1, W2, b2):
    nb = R // MB + 1

    def eidx(b):
        return jnp.minimum(b // NM, E - 1)

    return pl.pallas_call(
        _ffn_body,
        grid=(nb, NF),
        in_specs=[
            pl.BlockSpec((MB, D), lambda b, f: (b, 0)),
            pl.BlockSpec((1, FB, D), lambda b, f: (eidx(b), f, 0)),
            pl.BlockSpec((1, 1, FB), lambda b, f: (eidx(b) * NF + f, 0, 0)),
            pl.BlockSpec((1, D, FB), lambda b, f: (eidx(b), 0, f)),
            pl.BlockSpec((1, 1, D), lambda b, f: (eidx(b), 0, 0)),
            pl.BlockSpec((MB, 8), lambda b, f: (b, 0)),
        ],
        out_specs=pl.BlockSpec((MB, D), lambda b, f: (b, 0)),
        out_shape=jax.ShapeDtypeStruct((R + MB, D), jnp.float32),
        compiler_params=pltpu.CompilerParams(
            dimension_semantics=("parallel", "arbitrary")),
    )(xe, W1, b1.reshape(E * NF, 1, FB), W2, b2.reshape(E, 1, D), qxe[:, :8])


def _combine(outbuf, dst0, dst1):
    mesh = plsc.VectorSubcoreMesh(core_axis_name="c", subcore_axis_name="s")
    nw = mesh.num_cores * mesh.num_subcores
    per_w = N // nw
    ch = 32
    nch = per_w // ch
    nv = D // 16

    @functools.partial(
        pl.kernel,
        out_type=jax.ShapeDtypeStruct((N, D), jnp.float32),
        mesh=mesh,
        scratch_types=[
            pltpu.VMEM((ch,), jnp.int32),
            pltpu.VMEM((ch,), jnp.int32),
            pltpu.VMEM((ch, D), jnp.float32),
            pltpu.VMEM((ch, D), jnp.float32),
            pltpu.SemaphoreType.DMA,
            pltpu.SemaphoreType.DMA,
        ],
    )
    def comb(ob_hbm, d0_hbm, d1_hbm, y_hbm,
             idx0_v, idx1_v, r0_v, r1_v, sem0, sem1):
        wid = lax.axis_index("s") * mesh.num_cores + lax.axis_index("c")

        def body(ci, carry):
            base = pl.multiple_of(wid * per_w + ci * ch, ch)
            pltpu.sync_copy(d0_hbm.at[pl.ds(base, ch)], idx0_v)
            pltpu.sync_copy(d1_hbm.at[pl.ds(base, ch)], idx1_v)
            c0 = pltpu.async_copy(ob_hbm.at[idx0_v], r0_v, sem0)
            c1 = pltpu.async_copy(ob_hbm.at[idx1_v], r1_v, sem1)
            c0.wait()
            c1.wait()

            def tok(t, carry2):
                def vec(v, c):
                    sl = pl.ds(v * 16, 16)
                    r0_v[t, sl] = r0_v[t, sl] + r1_v[t, sl]
                    return c
                return lax.fori_loop(0, nv, vec, carry2, unroll=8)

            lax.fori_loop(0, ch, tok, 0)
            pltpu.sync_copy(r0_v, y_hbm.at[pl.ds(base, ch)])
            return carry

        lax.fori_loop(0, nch, body, 0)

    return comb(outbuf, dst0, dst1)


def kernel(x, noise, lw_w, lw_b, ln_w, ln_b, W1, b1, W2, b2):
    x2d = x.reshape(N, D)
    noise2d = noise.reshape(N, E)
    dst0, dst1, q0, q1 = _router(
        x2d, noise2d, lw_w.T, lw_b.reshape(1, E), ln_w.T, ln_b.reshape(1, E))
    xe, qxe = _dispatch(x2d, dst0, dst1, q0, q1)
    outbuf = _ffn(xe, qxe, W1, b1, W2, b2)
    y = _combine(outbuf, dst0, dst1)
    return y.reshape(B, T, D)


# speedup vs baseline: 1.1859x; 1.1312x over previous
import functools

import jax
import jax.numpy as jnp
from jax import lax
from jax.experimental import pallas as pl
from jax.experimental.pallas import tpu as pltpu
from jax.experimental.pallas import tpu_sc as plsc

B, T, D = 4, 2048, 1024
E, TOPK = 8, 2
N = B * T
CAP = N * TOPK // E
R = E * CAP
RPAD = 8
HID = 4 * D
QW = 128

TB = 1024
MB = 1024
FB = 2048
NM = CAP // MB
NF = HID // FB


def _router_body(x_ref, lwwT_ref, lwb_ref, lnwT_ref, lnb_ref, noise_ref,
                 dst0_ref, dst1_ref, q0_ref, q1_ref, cnt_ref):
    pid = pl.program_id(0)

    @pl.when(pid == 0)
    def _():
        cnt_ref[...] = jnp.zeros_like(cnt_ref)

    x = x_ref[...]
    logits = jnp.dot(x, lwwT_ref[...], preferred_element_type=jnp.float32)
    logits = logits + lwb_ref[...]
    zn = jnp.dot(x, lnwT_ref[...], preferred_element_type=jnp.float32)
    zn = zn + lnb_ref[...]
    sp = jnp.maximum(zn, 0.0) + jnp.log1p(jnp.exp(-jnp.abs(zn)))
    noisy = logits + noise_ref[...] * sp

    iota_e = lax.broadcasted_iota(jnp.int32, (TB, E), 1)
    v0 = jnp.max(noisy, axis=1, keepdims=True)
    e0 = jnp.min(jnp.where(noisy == v0, iota_e, E), axis=1, keepdims=True)
    oh0 = iota_e == e0
    noisy2 = jnp.where(oh0, -jnp.inf, noisy)
    v1 = jnp.max(noisy2, axis=1, keepdims=True)
    e1 = jnp.min(jnp.where(noisy2 == v1, iota_e, E), axis=1, keepdims=True)
    oh1 = iota_e == e1

    ex = jnp.exp(v1 - v0)
    denom = 1.0 + ex
    p0 = 1.0 / denom
    p1 = ex / denom

    m = (oh0 | oh1).astype(jnp.float32)
    ii = lax.broadcasted_iota(jnp.int32, (TB, TB), 0)
    jj = lax.broadcasted_iota(jnp.int32, (TB, TB), 1)
    lt = (jj < ii).astype(jnp.float32)
    excl = jnp.dot(lt, m, preferred_element_type=jnp.float32) + cnt_ref[...]
    pos0 = jnp.sum(jnp.where(oh0, excl, 0.0), axis=1, keepdims=True)
    pos1 = jnp.sum(jnp.where(oh1, excl, 0.0), axis=1, keepdims=True)
    cnt_ref[...] = cnt_ref[...] + jnp.sum(m, axis=0, keepdims=True)

    pos0 = pos0.astype(jnp.int32)
    pos1 = pos1.astype(jnp.int32)
    keep0 = pos0 < CAP
    keep1 = pos1 < CAP
    dst0_ref[...] = jnp.where(keep0, e0 * CAP + pos0, R)
    dst1_ref[...] = jnp.where(keep1, e1 * CAP + pos1, R)
    q0_ref[...] = jnp.where(keep0, p0, 0.0)
    q1_ref[...] = jnp.where(keep1, p1, 0.0)


def _router(x2d, noise2d, lwwT, lwb, lnwT, lnb):
    nblk = N // TB
    out = pl.pallas_call(
        _router_body,
        grid=(nblk,),
        in_specs=[
            pl.BlockSpec((TB, D), lambda i: (i, 0)),
            pl.BlockSpec((D, E), lambda i: (0, 0)),
            pl.BlockSpec((1, E), lambda i: (0, 0)),
            pl.BlockSpec((D, E), lambda i: (0, 0)),
            pl.BlockSpec((1, E), lambda i: (0, 0)),
            pl.BlockSpec((TB, E), lambda i: (i, 0)),
        ],
        out_specs=[
            pl.BlockSpec((TB, 1), lambda i: (i, 0)),
            pl.BlockSpec((TB, 1), lambda i: (i, 0)),
            pl.BlockSpec((TB, 1), lambda i: (i, 0)),
            pl.BlockSpec((TB, 1), lambda i: (i, 0)),
        ],
        out_shape=[
            jax.ShapeDtypeStruct((N, 1), jnp.int32),
            jax.ShapeDtypeStruct((N, 1), jnp.int32),
            jax.ShapeDtypeStruct((N, 1), jnp.float32),
            jax.ShapeDtypeStruct((N, 1), jnp.float32),
        ],
        scratch_shapes=[pltpu.VMEM((1, E), jnp.float32)],
    )(x2d, lwwT, lwb, lnwT, lnb, noise2d)
    dst0, dst1, q0, q1 = out
    return (dst0.reshape(N), dst1.reshape(N), q0.reshape(N), q1.reshape(N))


def _dispatch(x2d, dst0, dst1, q0, q1):
    mesh = plsc.VectorSubcoreMesh(core_axis_name="c", subcore_axis_name="s")
    nw = mesh.num_cores * mesh.num_subcores
    per_w = N // nw
    ch = 64
    nch = per_w // ch

    @functools.partial(
        pl.kernel,
        out_type=(
            jax.ShapeDtypeStruct((R + MB, D), jnp.float32),
            jax.ShapeDtypeStruct((R + MB, QW), jnp.float32),
        ),
        mesh=mesh,
        scratch_types=[
            pltpu.VMEM((ch,), jnp.int32),
            pltpu.VMEM((ch,), jnp.int32),
            pltpu.VMEM((ch,), jnp.float32),
            pltpu.VMEM((ch,), jnp.float32),
            pltpu.VMEM((ch, D), jnp.float32),
            pltpu.VMEM((ch, QW), jnp.float32),
            pltpu.VMEM((ch, QW), jnp.float32),
            pltpu.SemaphoreType.DMA,
            pltpu.SemaphoreType.DMA,
        ],
    )
    def disp(x_hbm, d0_hbm, d1_hbm, q0_hbm, q1_hbm, xe_hbm, qxe_hbm,
             idx0_v, idx1_v, q0_v, q1_v, xbuf, qb0, qb1, sem0, sem1):
        wid = lax.axis_index("s") * mesh.num_cores + lax.axis_index("c")

        def body(ci, carry):
            base = pl.multiple_of(wid * per_w + ci * ch, ch)
            pltpu.sync_copy(x_hbm.at[pl.ds(base, ch)], xbuf)
            pltpu.sync_copy(d0_hbm.at[pl.ds(base, ch)], idx0_v)
            pltpu.sync_copy(d1_hbm.at[pl.ds(base, ch)], idx1_v)
            pltpu.sync_copy(q0_hbm.at[pl.ds(base, ch)], q0_v)
            pltpu.sync_copy(q1_hbm.at[pl.ds(base, ch)], q1_v)
            for g in range(ch // 16):
                qv0 = q0_v[pl.ds(g * 16, 16)]
                qv1 = q1_v[pl.ds(g * 16, 16)]
                for t16 in range(16):
                    t = g * 16 + t16
                    v0b = jnp.full((16,), qv0[t16], jnp.float32)
                    v1b = jnp.full((16,), qv1[t16], jnp.float32)
                    for j in range(QW // 16):
                        qb0[t, pl.ds(j * 16, 16)] = v0b
                        qb1[t, pl.ds(j * 16, 16)] = v1b
            c0 = pltpu.async_copy(xbuf, xe_hbm.at[idx0_v], sem0)
            c1 = pltpu.async_copy(xbuf, xe_hbm.at[idx1_v], sem1)
            c2 = pltpu.async_copy(qb0, qxe_hbm.at[idx0_v], sem0)
            c3 = pltpu.async_copy(qb1, qxe_hbm.at[idx1_v], sem1)
            c0.wait()
            c1.wait()
            c2.wait()
            c3.wait()
            return carry

        lax.fori_loop(0, nch, body, 0)

    return disp(x2d, dst0, dst1, q0, q1)


def _ffn_body(xe_ref, w1_ref, b1_ref, w2_ref, b2_ref, q_ref, out_ref):
    f = pl.program_id(1)
    xb = xe_ref[...]
    h = lax.dot_general(xb, w1_ref[0], (((1,), (1,)), ((), ())),
                        preferred_element_type=jnp.float32)
    h = jnp.maximum(h + b1_ref[0], 0.0)
    part = lax.dot_general(h, w2_ref[0], (((1,), (1,)), ((), ())),
                           preferred_element_type=jnp.float32)

    @pl.when(f == 0)
    def _():
        out_ref[...] = part + b2_ref[0]

    @pl.when(jnp.logical_and(f > 0, f < NF - 1))
    def _():
        out_ref[...] = out_ref[...] + part

    @pl.when(f == NF - 1)
    def _():
        out_ref[...] = (out_ref[...] + part) * q_ref[:, 0:1]


def _ffn(xe, qxe, W1, b1, W2, b2):
    nb = R // MB + 1

    def eidx(b):
        return jnp.minimum(b // NM, E - 1)

    return pl.pallas_call(
        _ffn_body,
        grid=(nb, NF),
        in_specs=[
            pl.BlockSpec((MB, D), lambda b, f: (b, 0)),
            pl.BlockSpec((1, FB, D), lambda b, f: (eidx(b), f, 0)),
            pl.BlockSpec((1, 1, FB), lambda b, f: (eidx(b) * NF + f, 0, 0)),
            pl.BlockSpec((1, D, FB), lambda b, f: (eidx(b), 0, f)),
            pl.BlockSpec((1, 1, D), lambda b, f: (eidx(b), 0, 0)),
            pl.BlockSpec((MB, 8), lambda b, f: (b, 0)),
        ],
        out_specs=pl.BlockSpec((MB, D), lambda b, f: (b, 0)),
        out_shape=jax.ShapeDtypeStruct((R + MB, D), jnp.float32),
        compiler_params=pltpu.CompilerParams(
            dimension_semantics=("parallel", "arbitrary"),
            vmem_limit_bytes=100 * 1024 * 1024),
    )(xe, W1, b1.reshape(E * NF, 1, FB), W2, b2.reshape(E, 1, D), qxe[:, :8])


def _combine(outbuf, dst0, dst1):
    mesh = plsc.VectorSubcoreMesh(core_axis_name="c", subcore_axis_name="s")
    nw = mesh.num_cores * mesh.num_subcores
    per_w = N // nw
    ch = 32
    nch = per_w // ch
    nv = D // 16

    @functools.partial(
        pl.kernel,
        out_type=jax.ShapeDtypeStruct((N, D), jnp.float32),
        mesh=mesh,
        scratch_types=[
            pltpu.VMEM((ch,), jnp.int32),
            pltpu.VMEM((ch,), jnp.int32),
            pltpu.VMEM((ch, D), jnp.float32),
            pltpu.VMEM((ch, D), jnp.float32),
            pltpu.SemaphoreType.DMA,
            pltpu.SemaphoreType.DMA,
        ],
    )
    def comb(ob_hbm, d0_hbm, d1_hbm, y_hbm,
             idx0_v, idx1_v, r0_v, r1_v, sem0, sem1):
        wid = lax.axis_index("s") * mesh.num_cores + lax.axis_index("c")

        def body(ci, carry):
            base = pl.multiple_of(wid * per_w + ci * ch, ch)
            pltpu.sync_copy(d0_hbm.at[pl.ds(base, ch)], idx0_v)
            pltpu.sync_copy(d1_hbm.at[pl.ds(base, ch)], idx1_v)
            c0 = pltpu.async_copy(ob_hbm.at[idx0_v], r0_v, sem0)
            c1 = pltpu.async_copy(ob_hbm.at[idx1_v], r1_v, sem1)
            c0.wait()
            c1.wait()

            def tok(t, carry2):
                def vec(v, c):
                    sl = pl.ds(v * 16, 16)
                    r0_v[t, sl] = r0_v[t, sl] + r1_v[t, sl]
                    return c
                return lax.fori_loop(0, nv, vec, carry2, unroll=8)

            lax.fori_loop(0, ch, tok, 0)
            pltpu.sync_copy(r0_v, y_hbm.at[pl.ds(base, ch)])
            return carry

        lax.fori_loop(0, nch, body, 0)

    return comb(outbuf, dst0, dst1)


def kernel(x, noise, lw_w, lw_b, ln_w, ln_b, W1, b1, W2, b2):
    x2d = x.reshape(N, D)
    noise2d = noise.reshape(N, E)
    dst0, dst1, q0, q1 = _router(
        x2d, noise2d, lw_w.T, lw_b.reshape(1, E), ln_w.T, ln_b.reshape(1, E))
    xe, qxe = _dispatch(x2d, dst0, dst1, q0, q1)
    outbuf = _ffn(xe, qxe, W1, b1, W2, b2)
    y = _combine(outbuf, dst0, dst1)
    return y.reshape(B, T, D)
